# baseline (speedup 1.0000x reference)
import jax
import jax.numpy as jnp
from jax import lax
from jax.experimental import pallas as pl
from jax.experimental.pallas import tpu as pltpu
from jax.experimental.pallas import tpu_sc as plsc

NC = 2
NS = 16
L = 16
NW = NC * NS

B = 4096
F = 26
D = 16
FIELD_DIM = 100000

BPW = B // NW
KPW = BPW * F
NSTREAM = KPW // BPW


def _fm_body(x_hbm, wemb_hbm, wlin_hbm, bias_hbm, out_hbm,
             x_v, idx_v, rows_v, lin_v, out_v, bias_v,
             sem_e, sem_l):
    wid = lax.axis_index("s") * NC + lax.axis_index("c")
    base_k = wid * KPW

    pltpu.sync_copy(x_hbm.at[pl.ds(base_k, KPW)], x_v)
    pltpu.sync_copy(bias_hbm, bias_v)

    lane = lax.broadcasted_iota(jnp.int32, (L,), 0)

    def build_idx(m, carry):
        for st in range(BPW // L):
            pos = lane + (m * BPW + st * L)
            off = (pos % F) * FIELD_DIM
            idx_v[m, pl.ds(st * L, L)] = x_v[pl.ds(m * BPW + st * L, L)] + off
        return carry
    lax.fori_loop(0, NSTREAM, build_idx, 0)

    def fire(m, carry):
        pltpu.make_async_copy(
            wemb_hbm.at[idx_v.at[m]], rows_v.at[pl.ds(m * BPW, BPW)], sem_e
        ).start()
        pltpu.make_async_copy(
            wlin_hbm.at[idx_v.at[m]], lin_v.at[pl.ds(m * BPW, BPW)], sem_l
        ).start()
        return carry
    lax.fori_loop(0, NSTREAM, fire, 0)

    def drain(m, carry):
        pltpu.make_async_copy(
            wemb_hbm.at[idx_v.at[m]], rows_v.at[pl.ds(m * BPW, BPW)], sem_e
        ).wait()
        pltpu.make_async_copy(
            wlin_hbm.at[idx_v.at[m]], lin_v.at[pl.ds(m * BPW, BPW)], sem_l
        ).wait()
        return carry
    lax.fori_loop(0, NSTREAM, drain, 0)

    bias_vec = bias_v[...]
    lmask = lane < (F - L)

    def do_group(g, carry):
        out16 = jnp.zeros((L,), jnp.float32)
        for i in range(L):
            base = (g * L + i) * F
            s = jnp.zeros((L,), jnp.float32)
            ss = jnp.zeros((L,), jnp.float32)
            for j in range(F):
                v = rows_v[base + j, :]
                s = s + v
                ss = ss + v * v
            lina = lin_v[pl.ds(base, L)]
            linb = lin_v[pl.ds(base + L, L)]
            t = 0.5 * (s * s - ss) + lina + jnp.where(lmask, linb, 0.0)
            out16 = jnp.where(lane == i, jnp.sum(t), out16)
        out_v[pl.ds(g * L, L)] = out16 + bias_vec
        return carry
    lax.fori_loop(0, BPW // L, do_group, 0)

    pltpu.sync_copy(out_v, out_hbm.at[pl.ds(wid * BPW, BPW)])


@jax.jit
def _fm(x_flat, w_emb, w_lin_flat, bias16):
    run = pl.kernel(
        _fm_body,
        out_type=jax.ShapeDtypeStruct((B,), jnp.float32),
        mesh=plsc.VectorSubcoreMesh(core_axis_name="c", subcore_axis_name="s"),
        compiler_params=pltpu.CompilerParams(
            needs_layout_passes=False, use_tc_tiling_on_sc=False),
        scratch_types=[
            pltpu.VMEM((KPW,), jnp.int32),
            pltpu.VMEM((NSTREAM, BPW), jnp.int32),
            pltpu.VMEM((KPW, D), jnp.float32),
            pltpu.VMEM((KPW + L,), jnp.float32),
            pltpu.VMEM((BPW,), jnp.float32),
            pltpu.VMEM((L,), jnp.float32),
            pltpu.SemaphoreType.DMA,
            pltpu.SemaphoreType.DMA,
        ],
    )
    return run(x_flat, w_emb, w_lin_flat, bias16)


def kernel(x, W_emb, W_lin, bias):
    x_flat = x.astype(jnp.int32).reshape(B * F)
    w_lin_flat = W_lin.reshape(-1)
    bias16 = jnp.broadcast_to(bias.astype(jnp.float32), (L,))
    return _fm(x_flat, W_emb, w_lin_flat, bias16)
